# baseline (speedup 1.0000x reference)
import functools

import jax
import jax.numpy as jnp
from jax import lax
from jax.experimental import pallas as pl
from jax.experimental.pallas import tpu as pltpu
from jax.experimental.pallas import tpu_sc as plsc

_NC = 2
_NS = 16
_NW = _NC * _NS
_L = 16


@functools.partial(jax.jit, static_argnums=(3, 4))
def _sc_add_gather(x, idx, tab, n_rows, d):
    rows_per_w = n_rows // _NW
    C = 64
    n_chunks = rows_per_w // C
    mesh = plsc.VectorSubcoreMesh(core_axis_name="c", subcore_axis_name="s")

    @functools.partial(
        pl.kernel,
        out_type=jax.ShapeDtypeStruct((n_rows, d), jnp.float32),
        mesh=mesh,
        scratch_types=[
            pltpu.VMEM((C,), jnp.int32),
            pltpu.VMEM((C, d), jnp.float32),
            pltpu.VMEM((C, d), jnp.float32),
            pltpu.SemaphoreType.DMA,
            pltpu.SemaphoreType.DMA,
        ],
    )
    def k(x_hbm, idx_hbm, tab_hbm, out_hbm, idx_v, in_v, row_v, sem_in, sem_tab):
        wid = lax.axis_index("s") * _NC + lax.axis_index("c")
        base = wid * rows_per_w

        def chunk(ci, _):
            rb = base + ci * C
            cp_in = pltpu.async_copy(x_hbm.at[pl.ds(rb, C)], in_v, sem_in)
            pltpu.sync_copy(idx_hbm.at[pl.ds(rb, C)], idx_v)
            cp_tab = pltpu.async_copy(tab_hbm.at[idx_v], row_v, sem_tab)
            cp_in.wait()
            cp_tab.wait()

            def radd(r, _):
                for j in range(d // _L):
                    plsc.addupdate(
                        in_v.at[r, pl.ds(j * _L, _L)],
                        row_v[r, pl.ds(j * _L, _L)],
                    )
                return 0

            lax.fori_loop(0, C, radd, 0)
            pltpu.sync_copy(in_v, out_hbm.at[pl.ds(rb, C)])
            return 0

        lax.fori_loop(0, n_chunks, chunk, 0)

    return k(x, idx, tab)


def kernel(inputs, spatial_pos_grid_size, inputs_positions, pos_embedding):
    bs, T, d = inputs.shape
    n_rows = bs * T
    x = inputs.reshape(n_rows, d)
    idx = inputs_positions.reshape(n_rows).astype(jnp.int32)
    tab = pos_embedding[0]
    out = _sc_add_gather(x, idx, tab, n_rows, d)
    return out.reshape(bs, T, d)
